# baseline (speedup 1.0000x reference)
import jax
import jax.numpy as jnp
from jax.experimental import pallas as pl
from jax.experimental.pallas import tpu as pltpu

_BS = 512


def _body(pos_ref, kval_ref, vval_ref, kc_ref, vc_ref, ko_ref, vo_ref):
    j = pl.program_id(1)
    ko_ref[...] = kc_ref[...]
    vo_ref[...] = vc_ref[...]
    base = j * _BS
    q = kval_ref.shape[1]
    for i in range(q):
        p = pos_ref[i]
        off = p - base

        @pl.when((p >= base) & (p < base + _BS))
        def _():
            ko_ref[0, pl.ds(off, 1), :] = kval_ref[0, pl.ds(i, 1), :]
            vo_ref[0, pl.ds(off, 1), :] = vval_ref[0, pl.ds(i, 1), :]


def kernel(input_pos, k_val, v_val, k_cache, v_cache):
    B, S, H, D = k_cache.shape
    Q = k_val.shape[1]
    F = H * D
    kc = k_cache.reshape(B, S, F)
    vc = v_cache.reshape(B, S, F)
    kv = k_val.reshape(B, Q, F)
    vv = v_val.reshape(B, Q, F)
    grid = (B, S // _BS)
    out_k, out_v = pl.pallas_call(
        _body,
        grid=grid,
        in_specs=[
            pl.BlockSpec(memory_space=pltpu.SMEM),
            pl.BlockSpec((1, Q, F), lambda b, j: (b, 0, 0)),
            pl.BlockSpec((1, Q, F), lambda b, j: (b, 0, 0)),
            pl.BlockSpec((1, _BS, F), lambda b, j: (b, j, 0)),
            pl.BlockSpec((1, _BS, F), lambda b, j: (b, j, 0)),
        ],
        out_specs=[
            pl.BlockSpec((1, _BS, F), lambda b, j: (b, j, 0)),
            pl.BlockSpec((1, _BS, F), lambda b, j: (b, j, 0)),
        ],
        out_shape=[
            jax.ShapeDtypeStruct((B, S, F), jnp.float32),
            jax.ShapeDtypeStruct((B, S, F), jnp.float32),
        ],
        compiler_params=pltpu.CompilerParams(
            dimension_semantics=("parallel", "arbitrary")
        ),
    )(input_pos, kv, vv, kc, vc)
    return (out_k.reshape(B, S, H, D), out_v.reshape(B, S, H, D))


# speedup vs baseline: 1.0002x; 1.0002x over previous
import jax
import jax.numpy as jnp
from jax.experimental import pallas as pl
from jax.experimental.pallas import tpu as pltpu

_BS = 512


def _body(pos_ref, kval_ref, vval_ref, kc_ref, vc_ref, ko_ref, vo_ref):
    j = pl.program_id(1)
    ko_ref[...] = kc_ref[...]
    vo_ref[...] = vc_ref[...]
    base = j * _BS
    q = kval_ref.shape[1]
    hit = (pos_ref[0] >= base) & (pos_ref[0] < base + _BS)
    for i in range(1, q):
        hit |= (pos_ref[i] >= base) & (pos_ref[i] < base + _BS)

    @pl.when(hit)
    def _():
        for i in range(q):
            p = pos_ref[i]
            off = p - base

            @pl.when((p >= base) & (p < base + _BS))
            def _():
                ko_ref[0, pl.ds(off, 1), :] = kval_ref[0, pl.ds(i, 1), :]
                vo_ref[0, pl.ds(off, 1), :] = vval_ref[0, pl.ds(i, 1), :]


def kernel(input_pos, k_val, v_val, k_cache, v_cache):
    B, S, H, D = k_cache.shape
    Q = k_val.shape[1]
    F = H * D
    kc = k_cache.reshape(B, S, F)
    vc = v_cache.reshape(B, S, F)
    kv = k_val.reshape(B, Q, F)
    vv = v_val.reshape(B, Q, F)
    grid = (B, S // _BS)
    out_k, out_v = pl.pallas_call(
        _body,
        grid=grid,
        in_specs=[
            pl.BlockSpec(memory_space=pltpu.SMEM),
            pl.BlockSpec((1, Q, F), lambda b, j: (b, 0, 0)),
            pl.BlockSpec((1, Q, F), lambda b, j: (b, 0, 0)),
            pl.BlockSpec((1, _BS, F), lambda b, j: (b, j, 0)),
            pl.BlockSpec((1, _BS, F), lambda b, j: (b, j, 0)),
        ],
        out_specs=[
            pl.BlockSpec((1, _BS, F), lambda b, j: (b, j, 0)),
            pl.BlockSpec((1, _BS, F), lambda b, j: (b, j, 0)),
        ],
        out_shape=[
            jax.ShapeDtypeStruct((B, S, F), jnp.float32),
            jax.ShapeDtypeStruct((B, S, F), jnp.float32),
        ],
        compiler_params=pltpu.CompilerParams(
            dimension_semantics=("parallel", "arbitrary")
        ),
    )(input_pos, kv, vv, kc, vc)
    return (out_k.reshape(B, S, H, D), out_v.reshape(B, S, H, D))
